# baseline (speedup 1.0000x reference)
import functools

import jax
import jax.numpy as jnp
from jax import lax
from jax.experimental import pallas as pl
from jax.experimental.pallas import tpu as pltpu
from jax.experimental.pallas import tpu_sc as plsc

FIELD_H = 16
NCOLS = 26
NCOLS_PAD = 32
HIDDEN = 128

NC, NS = 2, 16
NW = NC * NS
SUB = 128
K = 16
SUPER = SUB * K


def _gather_body(nsup, table_hbm, idx_hbm, out_hbm,
                 idx_a, idx_b, rows_a, rows_b, sem_a, sem_b, wb_a, wb_b):
    wid = lax.axis_index("s") * NC + lax.axis_index("c")
    base = wid * nsup

    def fire(chunk, idx_v, rows_v, sem):
        off = (base + chunk) * SUPER
        pltpu.sync_copy(idx_hbm.at[pl.ds(off, SUPER)], idx_v)
        for j in range(K):
            pltpu.async_copy(
                table_hbm.at[idx_v.at[pl.ds(j * SUB, SUB)]],
                rows_v.at[pl.ds(j * SUB, SUB)],
                sem,
            )

    def drain_and_writeback(chunk, idx_v, rows_v, sem, wb):
        for j in range(K):
            pltpu.make_async_copy(
                table_hbm.at[idx_v.at[pl.ds(j * SUB, SUB)]],
                rows_v.at[pl.ds(j * SUB, SUB)],
                sem,
            ).wait()
        off = (base + chunk) * SUPER
        return pltpu.async_copy(rows_v, out_hbm.at[pl.ds(off, SUPER)], wb)

    def wb_wait(rows_v, wb):
        pltpu.make_async_copy(rows_v, out_hbm.at[pl.ds(0, SUPER)], wb).wait()

    fire(0, idx_a, rows_a, sem_a)
    fire(1, idx_b, rows_b, sem_b)

    def step(i, carry):
        g0 = 2 * i
        drain_and_writeback(g0, idx_a, rows_a, sem_a, wb_a)
        drain_and_writeback(g0 + 1, idx_b, rows_b, sem_b, wb_b)

        @pl.when(g0 + 2 < nsup)
        def _():
            wb_wait(rows_a, wb_a)
            fire(g0 + 2, idx_a, rows_a, sem_a)
            wb_wait(rows_b, wb_b)
            fire(g0 + 3, idx_b, rows_b, sem_b)

        return carry

    lax.fori_loop(0, nsup // 2, step, 0)
    wb_wait(rows_a, wb_a)
    wb_wait(rows_b, wb_b)


def _sc_gather(table, idx):
    n_idx = idx.shape[0]
    nsup = n_idx // (NW * SUPER)
    assert nsup * NW * SUPER == n_idx and nsup % 2 == 0
    mesh = plsc.VectorSubcoreMesh(core_axis_name="c", subcore_axis_name="s")
    kern = pl.kernel(
        functools.partial(_gather_body, nsup),
        out_type=jax.ShapeDtypeStruct((n_idx, FIELD_H), jnp.float32),
        mesh=mesh,
        scratch_types=[
            pltpu.VMEM((SUPER,), jnp.int32),
            pltpu.VMEM((SUPER,), jnp.int32),
            pltpu.VMEM((SUPER, FIELD_H), jnp.float32),
            pltpu.VMEM((SUPER, FIELD_H), jnp.float32),
            pltpu.SemaphoreType.DMA,
            pltpu.SemaphoreType.DMA,
            pltpu.SemaphoreType.DMA,
            pltpu.SemaphoreType.DMA,
        ],
        compiler_params=pltpu.CompilerParams(use_tc_tiling_on_sc=False),
    )
    return kern(table, idx)


def _mm_body(bb, seq, x_ref, w_ref, b_ref, o_ref):
    bm = bb * seq
    x3 = x_ref[...].reshape(bm, 4, HIDDEN)
    acc = lax.dot_general(
        x3[:, 0, :], w_ref[:, 0, :],
        (((1,), (1,)), ((), ())),
        preferred_element_type=jnp.float32,
    )
    for c in range(1, 4):
        acc = acc + lax.dot_general(
            x3[:, c, :], w_ref[:, c, :],
            (((1,), (1,)), ((), ())),
            preferred_element_type=jnp.float32,
        )
    o_ref[...] = (acc + b_ref[...]).reshape(bb, seq, HIDDEN)


def _tc_matmul(x128, w4, b2d, bsz, seq, bb):
    n128 = x128.shape[0]
    h = w4.shape[0]
    assert bsz % bb == 0 and n128 == bsz * seq * 4
    return pl.pallas_call(
        functools.partial(_mm_body, bb, seq),
        grid=(bsz // bb,),
        in_specs=[
            pl.BlockSpec((bb * seq * 4, HIDDEN), lambda i: (i, 0)),
            pl.BlockSpec((h, 4, HIDDEN), lambda i: (0, 0, 0)),
            pl.BlockSpec((1, h), lambda i: (0, 0)),
        ],
        out_specs=pl.BlockSpec((bb, seq, h), lambda i: (i, 0, 0)),
        out_shape=jax.ShapeDtypeStruct((bsz, seq, h), jnp.float32),
    )(x128, w4, b2d)


def kernel(input_ids, table, W, b):
    bsz, seq, ncols = input_ids.shape
    n_idx = bsz * seq * NCOLS_PAD
    cols = jnp.concatenate(
        [jnp.arange(ncols), jnp.arange(NCOLS_PAD - ncols)]
    ).astype(jnp.int32)
    idx = jnp.take(input_ids, cols, axis=2).reshape(n_idx)
    gathered = _sc_gather(table, idx)
    x128 = gathered.reshape(n_idx // 8, 8 * FIELD_H)
    w4 = jnp.pad(W, ((0, 0), (0, (NCOLS_PAD - NCOLS) * FIELD_H))).reshape(
        HIDDEN, 4, HIDDEN
    )
    return _tc_matmul(x128, w4, b.reshape(1, HIDDEN), bsz, seq, bb=64)


# speedup vs baseline: 1.0213x; 1.0213x over previous
import functools

import jax
import jax.numpy as jnp
from jax import lax
from jax.experimental import pallas as pl
from jax.experimental.pallas import tpu as pltpu
from jax.experimental.pallas import tpu_sc as plsc

FIELD_H = 16
NCOLS = 26
NCOLS_PAD = 32
HIDDEN = 128

NC, NS = 2, 16
NW = NC * NS
SUB = 128
K = 16
SUPER = SUB * K


def _gather_body(nsup, table_hbm, idx_hbm, out_hbm,
                 idx_a, idx_b, rows_a, rows_b, sem_a, sem_b, wb_a, wb_b):
    wid = lax.axis_index("s") * NC + lax.axis_index("c")
    base = wid * nsup

    def fire(chunk, idx_v, rows_v, sem):
        off = (base + chunk) * SUPER
        pltpu.sync_copy(idx_hbm.at[pl.ds(off, SUPER)], idx_v)
        for j in range(K):
            pltpu.async_copy(
                table_hbm.at[idx_v.at[pl.ds(j * SUB, SUB)]],
                rows_v.at[pl.ds(j * SUB, SUB)],
                sem,
            )

    def drain_and_writeback(chunk, idx_v, rows_v, sem, wb):
        for j in range(K):
            pltpu.make_async_copy(
                table_hbm.at[idx_v.at[pl.ds(j * SUB, SUB)]],
                rows_v.at[pl.ds(j * SUB, SUB)],
                sem,
            ).wait()
        off = (base + chunk) * SUPER
        return pltpu.async_copy(rows_v, out_hbm.at[pl.ds(off, SUPER)], wb)

    def wb_wait(rows_v, wb):
        pltpu.make_async_copy(rows_v, out_hbm.at[pl.ds(0, SUPER)], wb).wait()

    fire(0, idx_a, rows_a, sem_a)
    fire(1, idx_b, rows_b, sem_b)

    def step(i, carry):
        g0 = 2 * i
        drain_and_writeback(g0, idx_a, rows_a, sem_a, wb_a)
        drain_and_writeback(g0 + 1, idx_b, rows_b, sem_b, wb_b)

        @pl.when(g0 + 2 < nsup)
        def _():
            wb_wait(rows_a, wb_a)
            fire(g0 + 2, idx_a, rows_a, sem_a)
            wb_wait(rows_b, wb_b)
            fire(g0 + 3, idx_b, rows_b, sem_b)

        return carry

    lax.fori_loop(0, nsup // 2, step, 0)
    wb_wait(rows_a, wb_a)
    wb_wait(rows_b, wb_b)


def _sc_gather(table, idx):
    n_idx = idx.shape[0]
    nsup = n_idx // (NW * SUPER)
    assert nsup * NW * SUPER == n_idx and nsup % 2 == 0
    mesh = plsc.VectorSubcoreMesh(core_axis_name="c", subcore_axis_name="s")
    kern = pl.kernel(
        functools.partial(_gather_body, nsup),
        out_type=jax.ShapeDtypeStruct((n_idx, FIELD_H), jnp.float32),
        mesh=mesh,
        scratch_types=[
            pltpu.VMEM((SUPER,), jnp.int32),
            pltpu.VMEM((SUPER,), jnp.int32),
            pltpu.VMEM((SUPER, FIELD_H), jnp.float32),
            pltpu.VMEM((SUPER, FIELD_H), jnp.float32),
            pltpu.SemaphoreType.DMA,
            pltpu.SemaphoreType.DMA,
            pltpu.SemaphoreType.DMA,
            pltpu.SemaphoreType.DMA,
        ],
        compiler_params=pltpu.CompilerParams(use_tc_tiling_on_sc=False),
    )
    return kern(table, idx)


def _mm_body(bb, seq, x_ref, w_ref, b_ref, o_ref):
    bm = bb * seq
    x3 = x_ref[...].reshape(bm, 4, HIDDEN)
    acc = lax.dot_general(
        x3[:, 0, :], w_ref[:, 0, :],
        (((1,), (1,)), ((), ())),
        preferred_element_type=jnp.float32,
    )
    for c in range(1, 4):
        acc = acc + lax.dot_general(
            x3[:, c, :], w_ref[:, c, :],
            (((1,), (1,)), ((), ())),
            preferred_element_type=jnp.float32,
        )
    o_ref[...] = (acc + b_ref[...]).reshape(bb, seq, HIDDEN)


def _tc_matmul(x128, w4, b2d, bsz, seq, bb):
    n128 = x128.shape[0]
    h = w4.shape[0]
    assert bsz % bb == 0 and n128 == bsz * seq * 4
    return pl.pallas_call(
        functools.partial(_mm_body, bb, seq),
        grid=(bsz // bb,),
        in_specs=[
            pl.BlockSpec((bb * seq * 4, HIDDEN), lambda i: (i, 0)),
            pl.BlockSpec((h, 4, HIDDEN), lambda i: (0, 0, 0)),
            pl.BlockSpec((1, h), lambda i: (0, 0)),
        ],
        out_specs=pl.BlockSpec((bb, seq, h), lambda i: (i, 0, 0)),
        out_shape=jax.ShapeDtypeStruct((bsz, seq, h), jnp.float32),
    )(x128, w4, b2d)


def kernel(input_ids, table, W, b):
    bsz, seq, ncols = input_ids.shape
    n_idx = bsz * seq * NCOLS_PAD
    idx = jnp.pad(
        input_ids, ((0, 0), (0, 0), (0, NCOLS_PAD - ncols)), mode="wrap"
    ).reshape(n_idx)
    gathered = _sc_gather(table, idx)
    x128 = gathered.reshape(n_idx // 8, 8 * FIELD_H)
    w4 = jnp.pad(W, ((0, 0), (0, (NCOLS_PAD - NCOLS) * FIELD_H))).reshape(
        HIDDEN, 4, HIDDEN
    )
    return _tc_matmul(x128, w4, b.reshape(1, HIDDEN), bsz, seq, bb=128)
